# baseline (speedup 1.0000x reference)
import jax
import jax.numpy as jnp
from jax.experimental import pallas as pl
from jax.experimental.pallas import tpu as pltpu

N_ATOMS = 16384
N_STRUCT = 16
N_ELEM = 95
EMB = 64
HID = 256
XDIM = 128

BLOCK = 4096
HALF = 2048


def _fused_body(coord_ref, elems_ref, ind_ref, epad_ref, w1_ref, b1_ref,
                w2_ref, b2_ref, w3_ref, b3_ref, out_ref,
                wpad_ref, acc_ref, cnt_ref):
    i = pl.program_id(0)

    @pl.when(i == 0)
    def _init():
        wpad_ref[...] = jnp.dot(epad_ref[...], w1_ref[...],
                                preferred_element_type=jnp.float32)
        acc_ref[...] = jnp.zeros_like(acc_ref)
        cnt_ref[...] = jnp.zeros_like(cnt_ref)

    wpad = wpad_ref[...]
    w2 = w2_ref[...]
    b1 = b1_ref[0, :]
    b2 = b2_ref[0, :]

    for h in range(BLOCK // HALF):
        sl = pl.ds(h * HALF, HALF)
        elems = elems_ref[0, 0, sl]
        onehot = (jax.lax.broadcasted_iota(jnp.int32, (HALF, N_ELEM), 1)
                  == elems[:, None]).astype(jnp.float32)
        x = jnp.concatenate(
            [onehot, coord_ref[sl, :],
             jnp.zeros((HALF, XDIM - N_ELEM - 3), jnp.float32)], axis=1)
        hid = jnp.tanh(jnp.dot(x, wpad, preferred_element_type=jnp.float32)
                       + b1)
        hid = jnp.tanh(jnp.dot(hid, w2, preferred_element_type=jnp.float32)
                       + b2)
        ind = ind_ref[0, 0, sl]
        seg_t = (jax.lax.broadcasted_iota(jnp.int32, (N_STRUCT, HALF), 0)
                 == ind[None, :]).astype(jnp.float32)
        acc_ref[...] += jnp.dot(seg_t, hid,
                                preferred_element_type=jnp.float32)
        cnt_ref[...] += jnp.sum(seg_t, axis=1, keepdims=True)

    @pl.when(i == pl.num_programs(0) - 1)
    def _fin():
        out_ref[...] = (jnp.dot(acc_ref[...], w3_ref[...],
                                preferred_element_type=jnp.float32)
                        + b3_ref[0, 0] * cnt_ref[...])


@jax.jit
def kernel(coord, elems, ind_1, elem_embed, W1, b1, W2, b2, W3, b3):
    n = coord.shape[0]
    grid = n // BLOCK
    elems3 = elems.astype(jnp.int32).reshape(grid, 1, BLOCK)
    ind3 = ind_1.astype(jnp.int32).reshape(grid, 1, BLOCK)
    epad = jnp.concatenate([
        jnp.concatenate([elem_embed,
                         jnp.zeros((N_ELEM, 3), jnp.float32)], axis=1),
        jnp.concatenate([jnp.zeros((3, EMB), jnp.float32),
                         jnp.eye(3, dtype=jnp.float32)], axis=1),
        jnp.zeros((XDIM - N_ELEM - 3, EMB + 3), jnp.float32),
    ], axis=0)

    out = pl.pallas_call(
        _fused_body,
        grid=(grid,),
        in_specs=[
            pl.BlockSpec((BLOCK, 3), lambda i: (i, 0)),
            pl.BlockSpec((1, 1, BLOCK), lambda i: (i, 0, 0)),
            pl.BlockSpec((1, 1, BLOCK), lambda i: (i, 0, 0)),
            pl.BlockSpec((XDIM, EMB + 3), lambda i: (0, 0)),
            pl.BlockSpec((EMB + 3, HID), lambda i: (0, 0)),
            pl.BlockSpec((1, HID), lambda i: (0, 0)),
            pl.BlockSpec((HID, HID), lambda i: (0, 0)),
            pl.BlockSpec((1, HID), lambda i: (0, 0)),
            pl.BlockSpec((HID, 1), lambda i: (0, 0)),
            pl.BlockSpec((1, 1), lambda i: (0, 0)),
        ],
        out_specs=pl.BlockSpec((N_STRUCT, 1), lambda i: (0, 0)),
        out_shape=jax.ShapeDtypeStruct((N_STRUCT, 1), jnp.float32),
        scratch_shapes=[
            pltpu.VMEM((XDIM, HID), jnp.float32),
            pltpu.VMEM((N_STRUCT, HID), jnp.float32),
            pltpu.VMEM((N_STRUCT, 1), jnp.float32),
        ],
    )(coord, elems3, ind3, epad, W1, b1.reshape(1, HID), W2,
      b2.reshape(1, HID), W3, b3.reshape(1, 1))
    return out[:, 0]
